# Initial kernel scaffold; baseline (speedup 1.0000x reference)
import jax
import jax.numpy as jnp
from jax.experimental import pallas as pl


def kernel(x, W1, b1, W2, b2, Wk, bk, chart_queries, Wv, bv, codebook, Ws1, bs1, Ws2, bs2):
    raise NotImplementedError("write your pallas kernel here")


# speedup vs baseline: 7.1083x; 7.1083x over previous
import numpy as np
import jax
import jax.numpy as jnp
from jax.experimental import pallas as pl
from jax.experimental.pallas import tpu as pltpu

B = 4096
IN = 256
H = 768
D = 32
NC = 8
CPC = 128
SH = D // 2
BB = 512
NBLK = B // BB

_HI = jax.lax.Precision.HIGHEST


def _dot(a, b):
    return jax.lax.dot_general(a, b, (((1,), (0,)), ((), ())),
                               preferred_element_type=jnp.float32)


def _dotx(a, b):
    return jax.lax.dot_general(a, b, (((1,), (0,)), ((), ())),
                               precision=_HI, preferred_element_type=jnp.float32)


def _gelu(t):
    return t * (jax.lax.erf(t / np.sqrt(2).astype(np.float32)) + 1.0) / 2.0


def _fused_kernel(x_ref, w1_ref, b1_ref, w2_ref, b2_ref, wk_ref, bk_ref,
                  cq_ref, wv_ref, bv_ref, cb_ref, cbt_ref,
                  ws1_ref, bs1_ref, ws2_ref, bs2_ref,
                  kchart_ref, kcode_ref, zn_ref, ztex_ref, rw_ref, zgeo_ref,
                  vq_ref, idx_ref, znall_ref):
    x = x_ref[...]
    h1 = _gelu(_dot(x, w1_ref[...]) + b1_ref[...])
    feats = _gelu(_dot(h1, w2_ref[...]) + b2_ref[...])
    k = _dot(feats, wk_ref[...]) + bk_ref[...]
    scores = _dot(k, cq_ref[...]) / np.sqrt(float(H)).astype(np.float32)

    m = jnp.max(scores, axis=-1, keepdims=True)
    e = jnp.exp(scores - m)
    w = e / jnp.sum(e, axis=-1, keepdims=True)
    rw_ref[...] = w

    iota8 = jax.lax.broadcasted_iota(jnp.int32, (BB, NC), 1)
    wmax = jnp.max(w, axis=-1, keepdims=True)
    kchart = jnp.min(jnp.where(w == wmax, iota8, NC), axis=-1, keepdims=True)
    kchart_ref[...] = kchart

    v = _dot(feats, wv_ref[...]) + bv_ref[...]

    g = _dotx(v, cbt_ref[...])
    cn = jnp.sum(cbt_ref[...] * cbt_ref[...], axis=0)[None, :]
    t = cn - 2.0 * g
    iota128 = jax.lax.broadcasted_iota(jnp.int32, (BB, CPC), 1)

    zq_b = jnp.zeros((BB, D), dtype=jnp.float32)
    zn_b = jnp.zeros((BB, D), dtype=jnp.float32)
    kcode = jnp.zeros((BB, 1), dtype=jnp.int32)
    loss = jnp.zeros((1, 1), dtype=jnp.float32)
    for c in range(NC):
        tc = t[:, c * CPC:(c + 1) * CPC]
        tmin = jnp.min(tc, axis=-1, keepdims=True)
        idx_c = jnp.min(jnp.where(tc == tmin, iota128, CPC), axis=-1, keepdims=True)
        idx_ref[:, c:c + 1] = idx_c
        kcode = kcode + jnp.where(kchart == c, idx_c, 0)
        onehot = (iota128 == idx_c).astype(jnp.float32)
        zq_c = _dotx(onehot, cb_ref[c])
        w_c = w[:, c:c + 1]
        delta_c = v - zq_c
        loss = loss + jnp.sum(delta_c * delta_c * w_c, keepdims=True)
        zn_c = _dot(_gelu(_dot(delta_c, ws1_ref[...]) + bs1_ref[...]),
                    ws2_ref[...]) + bs2_ref[...]
        znall_ref[:, c, :] = zn_c
        zq_b = zq_b + zq_c * w_c
        zn_b = zn_b + zn_c * w_c

    kcode_ref[...] = kcode
    zn_ref[...] = zn_b
    ztex_ref[...] = (v - zq_b) - zn_b
    zgeo_ref[...] = (v + (zq_b - v)) + zn_b

    @pl.when(pl.program_id(0) == 0)
    def _init():
        vq_ref[...] = jnp.zeros((1, 1), dtype=jnp.float32)
    vq_ref[...] += loss


def kernel(x, W1, b1, W2, b2, Wk, bk, chart_queries, Wv, bv, codebook,
           Ws1, bs1, Ws2, bs2):
    cbt = codebook.reshape(NC * CPC, D).T
    full = lambda *shape: pl.BlockSpec(shape, lambda i: (0,) * len(shape))
    row = lambda *shape: pl.BlockSpec(shape, lambda i: (i,) + (0,) * (len(shape) - 1))

    out_shapes = (
        jax.ShapeDtypeStruct((B, 1), jnp.int32),
        jax.ShapeDtypeStruct((B, 1), jnp.int32),
        jax.ShapeDtypeStruct((B, D), jnp.float32),
        jax.ShapeDtypeStruct((B, D), jnp.float32),
        jax.ShapeDtypeStruct((B, NC), jnp.float32),
        jax.ShapeDtypeStruct((B, D), jnp.float32),
        jax.ShapeDtypeStruct((1, 1), jnp.float32),
        jax.ShapeDtypeStruct((B, NC), jnp.int32),
        jax.ShapeDtypeStruct((B, NC, D), jnp.float32),
    )
    in_specs = [
        row(BB, IN),
        full(IN, H), full(1, H), full(H, H), full(1, H), full(H, H), full(1, H),
        full(H, NC), full(H, D), full(1, D), full(NC, CPC, D), full(D, NC * CPC),
        full(D, SH), full(1, SH), full(SH, D), full(1, D),
    ]
    out_specs = (
        row(BB, 1), row(BB, 1), row(BB, D), row(BB, D), row(BB, NC), row(BB, D),
        full(1, 1), row(BB, NC), row(BB, NC, D),
    )
    outs = pl.pallas_call(
        _fused_kernel,
        grid=(NBLK,),
        in_specs=in_specs,
        out_specs=out_specs,
        out_shape=out_shapes,
    )(x, W1, b1[None, :], W2, b2[None, :], Wk, bk[None, :],
      chart_queries.T, Wv, bv[None, :], codebook, cbt,
      Ws1, bs1[None, :], Ws2, bs2[None, :])

    kchart, kcode, z_n, z_tex, rw, z_geo, vq, idx, znall = outs
    vq_loss = vq[0, 0] * np.float32(1.25 / (B * D))
    return (kchart[:, 0], kcode[:, 0], z_n, z_tex, rw, z_geo, vq_loss, idx, znall)
